# Initial kernel scaffold; baseline (speedup 1.0000x reference)
import jax
import jax.numpy as jnp
from jax.experimental import pallas as pl


def kernel(boxes, scores):
    raise NotImplementedError("write your pallas kernel here")


# speedup vs baseline: 109.7953x; 109.7953x over previous
import functools

import jax
import jax.numpy as jnp
from jax.experimental import pallas as pl
from jax.experimental.pallas import tpu as pltpu

_K = 4096
_NPOST = 500
_NMS_THRESH = 0.25
_SCORE_THRESH = 0.1
_TR = 512


def _nms_keep_body(b_ref, bt_ref, v_ref, vcol_ref, keep_ref, adj_ref):
    col = jax.lax.broadcasted_iota(jnp.int32, (_TR, _K), 1)

    def build_tile(t, carry):
        ts = t * _TR
        x1r = b_ref[pl.ds(ts, _TR), 0:1]
        y1r = b_ref[pl.ds(ts, _TR), 1:2]
        x2r = b_ref[pl.ds(ts, _TR), 2:3]
        y2r = b_ref[pl.ds(ts, _TR), 3:4]
        x1c = bt_ref[0:1, :]
        y1c = bt_ref[1:2, :]
        x2c = bt_ref[2:3, :]
        y2c = bt_ref[3:4, :]
        xx1 = jnp.maximum(x1r, x1c)
        yy1 = jnp.maximum(y1r, y1c)
        xx2 = jnp.minimum(x2r, x2c)
        yy2 = jnp.minimum(y2r, y2c)
        w = jnp.clip(xx2 - xx1, 0.0, None)
        h = jnp.clip(yy2 - yy1, 0.0, None)
        inter = w * h
        area_r = (x2r - x1r) * (y2r - y1r)
        area_c = (x2c - x1c) * (y2c - y1c)
        iou = inter / (area_r + area_c - inter + 1e-8)
        row = jax.lax.broadcasted_iota(jnp.int32, (_TR, _K), 0) + ts
        vrow = vcol_ref[pl.ds(ts, _TR), 0:1] > 0.0
        adj = (iou > _NMS_THRESH) & (col > row) & vrow
        adj_ref[pl.ds(ts, _TR), :] = jnp.where(adj, 1.0, 0.0).astype(jnp.bfloat16)
        return carry

    jax.lax.fori_loop(0, _K // _TR, build_tile, 0)

    v = v_ref[0:1, :] > 0.0

    def cond(carry):
        _, changed = carry
        return changed

    def body(carry):
        k, _ = carry
        m = jnp.dot(k.astype(jnp.bfloat16), adj_ref[...],
                    preferred_element_type=jnp.float32)
        nk = jnp.where(v & (m < 0.5), 1.0, 0.0)
        changed = jnp.sum(jnp.abs(nk - k)) > 0.0
        return nk, changed

    k0 = jnp.where(v, 1.0, 0.0)
    kfin, _ = jax.lax.while_loop(cond, body, (k0, jnp.bool_(True)))
    keep_ref[0:1, :] = kfin


def _nms_keep(b, bt, v_row, v_col):
    return pl.pallas_call(
        _nms_keep_body,
        out_shape=jax.ShapeDtypeStruct((1, _K), jnp.float32),
        scratch_shapes=[pltpu.VMEM((_K, _K), jnp.bfloat16)],
    )(b, bt, v_row, v_col)


def kernel(boxes, scores):
    probs = jax.nn.sigmoid(scores)
    valid = probs >= _SCORE_THRESH
    masked = jnp.where(valid, probs, -jnp.inf)
    _, idx = jax.lax.top_k(masked, _K)
    b = boxes[idx]
    s = probs[idx]
    v = valid[idx]
    vf = v.astype(jnp.float32)
    keep_f = _nms_keep(b, b.T, vf.reshape(1, _K), vf.reshape(_K, 1))
    keep = keep_f.reshape(_K) > 0.5
    kept_scores = jnp.where(keep, s, -jnp.inf)
    _, fidx = jax.lax.top_k(kept_scores, _NPOST)
    fkeep = keep[fidx].astype(s.dtype)
    final_boxes = b[fidx] * fkeep[:, None]
    final_scores = s[fidx] * fkeep
    return jnp.concatenate([final_boxes, final_scores[:, None]], axis=-1)
